# baseline (speedup 1.0000x reference)
import functools

import jax
import jax.numpy as jnp
import numpy as np
from jax import lax
from jax.experimental import pallas as pl
from jax.experimental.pallas import tpu as pltpu
from jax.experimental.pallas import tpu_sc as plsc

_FIELD_DIM = 38461
_NUM_FIELDS = 26
_EMBED = 16
_ROWS = 16384
_B = _ROWS * _NUM_FIELDS
_NC, _NS, _LANES = 2, 16, 16
_NW = _NC * _NS
_BPW = _B // _NW
_G = 128
_NG = _BPW // _G
_CHUNK = 1664
_GPC = _CHUNK // _G
_NCHUNK = _BPW // _CHUNK

_OFF_PATTERN = ((np.arange(_GPC * _G) % _NUM_FIELDS) * _FIELD_DIM).astype(
    np.int32).reshape(_GPC, _G)


def _body(x_hbm, w_hbm, off_hbm, out_hbm, idx_v, off_v, rows_v, gsem):
    cid = lax.axis_index("c")
    sid = lax.axis_index("s")
    wid = sid * _NC + cid

    pltpu.sync_copy(x_hbm.at[pl.ds(wid * _NG, _NG)], idx_v)
    pltpu.sync_copy(off_hbm, off_v)

    def add_row(g, carry):
        r = lax.rem(g, _GPC)
        for c in range(_G // _LANES):
            s = pl.ds(c * _LANES, _LANES)
            idx_v[g, s] = idx_v[g, s] + off_v[r, s]
        return carry

    lax.fori_loop(0, _NG, add_row, 0, unroll=False)

    base = wid * _BPW
    for chunk in range(_NCHUNK):
        def fire(g, carry):
            pltpu.async_copy(
                w_hbm.at[idx_v.at[chunk * _GPC + g]],
                rows_v.at[pl.ds(g * _G, _G)],
                gsem,
            )
            return carry

        lax.fori_loop(0, _GPC, fire, 0, unroll=False)
        out_slice = out_hbm.at[pl.ds(base + chunk * _CHUNK, _CHUNK)]
        pltpu.make_async_copy(out_slice, rows_v, gsem).wait()
        pltpu.sync_copy(rows_v, out_slice)


@jax.jit
def _embed(x2d, w, off):
    fn = pl.kernel(
        _body,
        out_type=jax.ShapeDtypeStruct((_B, _EMBED), jnp.float32),
        mesh=plsc.VectorSubcoreMesh(
            core_axis_name="c", subcore_axis_name="s",
            num_cores=_NC, num_subcores=_NS),
        scratch_types=[
            pltpu.VMEM((_NG, _G), jnp.int32),
            pltpu.VMEM((_GPC, _G), jnp.int32),
            pltpu.VMEM((_CHUNK, _EMBED), jnp.float32),
            pltpu.SemaphoreType.DMA,
        ],
        compiler_params=pltpu.CompilerParams(use_tc_tiling_on_sc=False),
    )
    return fn(x2d, w, off)


def kernel(x, W):
    x2d = x.reshape(_B // _G, _G).astype(jnp.int32)
    off = jnp.asarray(_OFF_PATTERN)
    out = _embed(x2d, W, off)
    return out.reshape(_ROWS, _NUM_FIELDS, _EMBED)


# speedup vs baseline: 1.6248x; 1.6248x over previous
import functools

import jax
import jax.numpy as jnp
import numpy as np
from jax import lax
from jax.experimental import pallas as pl
from jax.experimental.pallas import tpu as pltpu
from jax.experimental.pallas import tpu_sc as plsc

_FIELD_DIM = 38461
_NF = 26
_E = 16
_ROWS = 16384
_NC, _NS, _L = 2, 16, 16
_NW = _NC * _NS
_BW = _ROWS // _NW
_G = 128
_GPB = _BW // _G
_PLANE = _E * _ROWS
_HALF = _PLANE // 2
_BLK = _BW * 8


def _body(xt_hbm, w_hbm, out_hbm, idx_v, rows_a, rows_b, t5_a, t5_b,
          gsem, osem):
    cid = lax.axis_index("c")
    sid = lax.axis_index("s")
    wid = sid * _NC + cid
    b0 = wid * _BW

    pltpu.sync_copy(xt_hbm.at[:, pl.ds(b0, _BW)], idx_v)
    for f in range(_NF):
        def addoff(j, carry, f=f):
            s = pl.ds(j * _L, _L)
            idx_v[f, s] = idx_v[f, s] + (f * _FIELD_DIM)
            return carry
        lax.fori_loop(0, _BW // _L, addoff, 0, unroll=4)

    lanes = lax.iota(jnp.int32, _L)
    koff = (lanes >> 3) * _BLK + (lanes & 7) * 128

    rows_bufs = (rows_a, rows_b)
    t5_bufs = (t5_a, t5_b)

    def fire(f, rows_v):
        for g in range(_GPB):
            pltpu.async_copy(
                w_hbm.at[idx_v.at[f, pl.ds(g * _G, _G)]],
                rows_v.at[pl.ds(g * _G, _G)], gsem)

    fire(0, rows_bufs[0])
    for f in range(_NF):
        rows_v = rows_bufs[f % 2]
        t5_v = t5_bufs[f % 2]
        pltpu.make_async_copy(
            w_hbm.at[pl.ds(0, _BW)], rows_v, gsem).wait()
        if f + 1 < _NF:
            fire(f + 1, rows_bufs[(f + 1) % 2])
        if f >= 2:
            for g in range(2):
                pltpu.make_async_copy(
                    t5_v.at[pl.ds(g * _BLK, _BLK)],
                    out_hbm.at[f - 2, pl.ds(g * _HALF + wid * _BLK, _BLK)],
                    osem).wait()

        def tpose(j, carry):
            joff = (j >> 7) * 1024 + (j & 127)
            plsc.store_scatter(t5_v, [koff + joff], rows_v[j, :])
            return carry
        lax.fori_loop(0, _BW, tpose, 0, unroll=8)

        for g in range(2):
            pltpu.async_copy(
                t5_v.at[pl.ds(g * _BLK, _BLK)],
                out_hbm.at[f, pl.ds(g * _HALF + wid * _BLK, _BLK)], osem)

    for f in (_NF - 2, _NF - 1):
        for g in range(2):
            pltpu.make_async_copy(
                t5_bufs[f % 2].at[pl.ds(g * _BLK, _BLK)],
                out_hbm.at[f, pl.ds(g * _HALF + wid * _BLK, _BLK)],
                osem).wait()


@jax.jit
def _embed(xt, w):
    fn = pl.kernel(
        _body,
        out_type=jax.ShapeDtypeStruct((_NF, _PLANE), jnp.float32),
        mesh=plsc.VectorSubcoreMesh(
            core_axis_name="c", subcore_axis_name="s",
            num_cores=_NC, num_subcores=_NS),
        scratch_types=[
            pltpu.VMEM((_NF, _BW), jnp.int32),
            pltpu.VMEM((_BW, _E), jnp.float32),
            pltpu.VMEM((_BW, _E), jnp.float32),
            pltpu.VMEM((2 * _BLK,), jnp.float32),
            pltpu.VMEM((2 * _BLK,), jnp.float32),
            pltpu.SemaphoreType.DMA,
            pltpu.SemaphoreType.DMA,
        ],
        compiler_params=pltpu.CompilerParams(
            use_tc_tiling_on_sc=False, needs_layout_passes=False),
    )
    return fn(xt, w)


def kernel(x, W):
    xt = x.T.astype(jnp.int32)
    out2 = _embed(xt, W)
    out5 = out2.reshape(_NF, 2, _ROWS // 128, 8, 128)
    return jnp.transpose(out5, (2, 4, 0, 1, 3)).reshape(_ROWS, _NF, _E)
